# baseline (device time: 106932 ns/iter reference)
import functools

import jax
import jax.numpy as jnp
from jax import lax
from jax.experimental import pallas as pl
from jax.experimental.pallas import tpu as pltpu

N_DEV = 16
P = 4
J = 4
RA = 128
RB = 32
BOT = 512
NSUB = 2


def kernel(dy, W):
    m, k = dy.shape
    d, _ = W.shape
    sa = RA // NSUB
    sb = RB // NSUB

    def body(dy_ref, w_ref, out_ref,
             abufR, abufL, bbufR, bbufL,
             ssAR, ssAL, ssBR, ssBL,
             semAR, semAL, semBR, semBL):
        me = lax.axis_index("i")
        p = me // J
        j = me % J
        nbr_a = (p * J + (j + 1) % J, p * J + (j - 1) % J)
        nbr_b = (((p + 1) % P) * J + j, ((p - 1) % P) * J + j)

        def partial_dot(off, rows):
            return lax.dot_general(
                dy_ref[pl.ds(off, rows), :], w_ref[...],
                dimension_numbers=(((1,), (1,)), ((), ())),
                preferred_element_type=jnp.float32,
            )

        def rs_phase(nsteps, srows, send_off, recv_off, nbr,
                     bufs, ssems, rsems, acc):
            live = {}

            def start(dirn, s, h):
                off = send_off(dirn, s) + h * srows
                r = pltpu.make_async_remote_copy(
                    src_ref=out_ref.at[pl.ds(off, srows), :],
                    dst_ref=bufs[dirn].at[s, h],
                    send_sem=ssems[dirn].at[h],
                    recv_sem=rsems[dirn].at[s, h],
                    device_id=(nbr[dirn],),
                    device_id_type=pl.DeviceIdType.MESH,
                )
                r.start()
                live[(dirn, s, h)] = r

            for h in range(NSUB):
                start(0, 0, h)
                start(1, 0, h)
            for s in range(nsteps):
                for h in range(NSUB):
                    for dirn in (0, 1):
                        r = live[(dirn, s, h)]
                        r.wait_recv()
                        off = recv_off(dirn, s) + h * srows
                        acc(dirn, s, h, off)
                        r.wait_send()
                        if s + 1 < nsteps:
                            start(dirn, s + 1, h)

        def ag_phase(nsteps, srows, send_off, nbr, ssems, rsems, base):
            live = {}

            def start(dirn, s, h):
                off = send_off(dirn, s) + h * srows
                r = pltpu.make_async_remote_copy(
                    src_ref=out_ref.at[pl.ds(off, srows), :],
                    dst_ref=out_ref.at[pl.ds(off, srows), :],
                    send_sem=ssems[dirn].at[h],
                    recv_sem=rsems[dirn].at[base + s, h],
                    device_id=(nbr[dirn],),
                    device_id_type=pl.DeviceIdType.MESH,
                )
                r.start()
                live[(dirn, s, h)] = r

            for h in range(NSUB):
                start(0, 0, h)
                start(1, 0, h)
            for s in range(nsteps):
                for h in range(NSUB):
                    for dirn in (0, 1):
                        r = live[(dirn, s, h)]
                        r.wait_recv()
                        r.wait_send()
                        if s + 1 < nsteps:
                            start(dirn, s + 1, h)

        out_ref[pl.ds(j * RA, RA), :] = partial_dot(j * RA, RA)
        out_ref[pl.ds(BOT + j * RA, RA), :] = partial_dot(BOT + j * RA, RA)

        barrier_sem = pltpu.get_barrier_semaphore()
        for nbr in (*nbr_a, *nbr_b):
            pl.semaphore_signal(
                barrier_sem, inc=1,
                device_id=(nbr,), device_id_type=pl.DeviceIdType.MESH,
            )
        pl.semaphore_wait(barrier_sem, 4)

        rs_phase(
            J - 1, sa,
            lambda dirn, s: (((j - s) % J) * RA if dirn == 0
                             else BOT + ((j + s) % J) * RA),
            lambda dirn, s: (((j - s - 1) % J) * RA if dirn == 0
                             else BOT + ((j + s + 1) % J) * RA),
            nbr_a, (abufR, abufL), (ssAR, ssAL), (semAR, semAL),
            lambda dirn, s, h, off: out_ref.__setitem__(
                (pl.ds(off, sa), slice(None)),
                partial_dot(off, sa) + (abufR, abufL)[dirn][s, h],
            ),
        )

        top = ((j + 1) % J) * RA
        bot = BOT + ((j - 1) % J) * RA

        rs_phase(
            P - 1, sb,
            lambda dirn, s: (top + ((p - s) % P) * RB if dirn == 0
                             else bot + ((p + s) % P) * RB),
            lambda dirn, s: (top + ((p - s - 1) % P) * RB if dirn == 0
                             else bot + ((p + s + 1) % P) * RB),
            nbr_b, (bbufR, bbufL), (ssBR, ssBL), (semBR, semBL),
            lambda dirn, s, h, off: out_ref.__setitem__(
                (pl.ds(off, sb), slice(None)),
                out_ref[pl.ds(off, sb), :] + (bbufR, bbufL)[dirn][s, h],
            ),
        )

        ag_phase(
            P - 1, sb,
            lambda dirn, s: (top + ((p + 1 - s) % P) * RB if dirn == 0
                             else bot + ((p - 1 + s) % P) * RB),
            nbr_b, (ssBR, ssBL), (semBR, semBL), P - 1,
        )

        ag_phase(
            J - 1, sa,
            lambda dirn, s: (((j + 1 - s) % J) * RA if dirn == 0
                             else BOT + ((j - 1 + s) % J) * RA),
            nbr_a, (ssAR, ssAL), (semAR, semAL), J - 1,
        )

        @functools.partial(
            pl.run_scoped, exit_sem=pltpu.SemaphoreType.REGULAR
        )
        def _(exit_sem):
            for nbr in (*nbr_a, *nbr_b):
                pl.semaphore_signal(
                    exit_sem, inc=1,
                    device_id=(nbr,), device_id_type=pl.DeviceIdType.MESH,
                )
            pl.semaphore_wait(exit_sem, 4)

    return pl.pallas_call(
        body,
        out_shape=jax.ShapeDtypeStruct((m, d), jnp.float32),
        in_specs=[
            pl.BlockSpec(memory_space=pltpu.VMEM),
            pl.BlockSpec(memory_space=pltpu.VMEM),
        ],
        out_specs=pl.BlockSpec(memory_space=pltpu.VMEM),
        scratch_shapes=[
            pltpu.VMEM((J - 1, NSUB, sa, d), jnp.float32),
            pltpu.VMEM((J - 1, NSUB, sa, d), jnp.float32),
            pltpu.VMEM((P - 1, NSUB, sb, d), jnp.float32),
            pltpu.VMEM((P - 1, NSUB, sb, d), jnp.float32),
            pltpu.SemaphoreType.DMA((NSUB,)),
            pltpu.SemaphoreType.DMA((NSUB,)),
            pltpu.SemaphoreType.DMA((NSUB,)),
            pltpu.SemaphoreType.DMA((NSUB,)),
            pltpu.SemaphoreType.DMA((2 * (J - 1), NSUB)),
            pltpu.SemaphoreType.DMA((2 * (J - 1), NSUB)),
            pltpu.SemaphoreType.DMA((2 * (P - 1), NSUB)),
            pltpu.SemaphoreType.DMA((2 * (P - 1), NSUB)),
        ],
        compiler_params=pltpu.CompilerParams(collective_id=0),
    )(dy, W)


# device time: 89773 ns/iter; 1.1911x vs baseline; 1.1911x over previous
import functools

import jax
import jax.numpy as jnp
from jax import lax
from jax.experimental import pallas as pl
from jax.experimental.pallas import tpu as pltpu

N_DEV = 16
P = 4
J = 4
RA = 128
RB = 32
BOT = 512
NSUB = 2


def kernel(dy, W):
    m, k = dy.shape
    d, _ = W.shape
    sa = RA // NSUB
    sb = RB // NSUB

    def body(dy_ref, w_ref, out_ref,
             abufR, abufL, bbufR, bbufL,
             ssAR, ssAL, ssBR, ssBL,
             semAR, semAL, semBR, semBL):
        me = lax.axis_index("i")
        p = me // J
        j = me % J
        nbr_a = (p * J + (j + 1) % J, p * J + (j - 1) % J)
        nbr_b = (((p + 1) % P) * J + j, ((p - 1) % P) * J + j)

        def gemm(off):
            out_ref[pl.ds(off, RA), :] = lax.dot_general(
                dy_ref[pl.ds(off, RA), :], w_ref[...],
                dimension_numbers=(((1,), (1,)), ((), ())),
                preferred_element_type=jnp.float32,
            )

        def rs_phase(nsteps, srows, send_off, recv_off, nbr,
                     bufs, ssems, rsems, hook):
            live = {}

            def start(dirn, s, h):
                off = send_off(dirn, s) + h * srows
                r = pltpu.make_async_remote_copy(
                    src_ref=out_ref.at[pl.ds(off, srows), :],
                    dst_ref=bufs[dirn].at[s, h],
                    send_sem=ssems[dirn].at[h],
                    recv_sem=rsems[dirn].at[s, h],
                    device_id=(nbr[dirn],),
                    device_id_type=pl.DeviceIdType.MESH,
                )
                r.start()
                live[(dirn, s, h)] = r

            for h in range(NSUB):
                start(0, 0, h)
                start(1, 0, h)
            hook(0)
            for s in range(nsteps):
                for h in range(NSUB):
                    for dirn in (0, 1):
                        r = live[(dirn, s, h)]
                        r.wait_recv()
                        off = recv_off(dirn, s) + h * srows
                        out_ref[pl.ds(off, srows), :] += bufs[dirn][s, h]
                        r.wait_send()
                        if s + 1 < nsteps:
                            start(dirn, s + 1, h)
                if s + 1 < nsteps:
                    hook(s + 1)

        def ag_phase(nsteps, srows, send_off, nbr, ssems, rsems, base):
            live = {}

            def start(dirn, s, h):
                off = send_off(dirn, s) + h * srows
                r = pltpu.make_async_remote_copy(
                    src_ref=out_ref.at[pl.ds(off, srows), :],
                    dst_ref=out_ref.at[pl.ds(off, srows), :],
                    send_sem=ssems[dirn].at[h],
                    recv_sem=rsems[dirn].at[base + s, h],
                    device_id=(nbr[dirn],),
                    device_id_type=pl.DeviceIdType.MESH,
                )
                r.start()
                live[(dirn, s, h)] = r

            for h in range(NSUB):
                start(0, 0, h)
                start(1, 0, h)
            for s in range(nsteps):
                for h in range(NSUB):
                    for dirn in (0, 1):
                        r = live[(dirn, s, h)]
                        r.wait_recv()
                        r.wait_send()
                        if s + 1 < nsteps:
                            start(dirn, s + 1, h)

        gemm(j * RA)
        gemm(BOT + j * RA)

        barrier_sem = pltpu.get_barrier_semaphore()
        for nbr in (*nbr_a, *nbr_b):
            pl.semaphore_signal(
                barrier_sem, inc=1,
                device_id=(nbr,), device_id_type=pl.DeviceIdType.MESH,
            )
        pl.semaphore_wait(barrier_sem, 4)

        rs_phase(
            J - 1, sa,
            lambda dirn, s: (((j - s) % J) * RA if dirn == 0
                             else BOT + ((j + s) % J) * RA),
            lambda dirn, s: (((j - s - 1) % J) * RA if dirn == 0
                             else BOT + ((j + s + 1) % J) * RA),
            nbr_a, (abufR, abufL), (ssAR, ssAL), (semAR, semAL),
            lambda s: (gemm(((j - s - 1) % J) * RA),
                       gemm(BOT + ((j + s + 1) % J) * RA)),
        )

        top = ((j + 1) % J) * RA
        bot = BOT + ((j - 1) % J) * RA

        rs_phase(
            P - 1, sb,
            lambda dirn, s: (top + ((p - s) % P) * RB if dirn == 0
                             else bot + ((p + s) % P) * RB),
            lambda dirn, s: (top + ((p - s - 1) % P) * RB if dirn == 0
                             else bot + ((p + s + 1) % P) * RB),
            nbr_b, (bbufR, bbufL), (ssBR, ssBL), (semBR, semBL),
            lambda s: None,
        )

        ag_phase(
            P - 1, sb,
            lambda dirn, s: (top + ((p + 1 - s) % P) * RB if dirn == 0
                             else bot + ((p - 1 + s) % P) * RB),
            nbr_b, (ssBR, ssBL), (semBR, semBL), P - 1,
        )

        ag_phase(
            J - 1, sa,
            lambda dirn, s: (((j + 1 - s) % J) * RA if dirn == 0
                             else BOT + ((j - 1 + s) % J) * RA),
            nbr_a, (ssAR, ssAL), (semAR, semAL), J - 1,
        )

        @functools.partial(
            pl.run_scoped, exit_sem=pltpu.SemaphoreType.REGULAR
        )
        def _(exit_sem):
            for nbr in (*nbr_a, *nbr_b):
                pl.semaphore_signal(
                    exit_sem, inc=1,
                    device_id=(nbr,), device_id_type=pl.DeviceIdType.MESH,
                )
            pl.semaphore_wait(exit_sem, 4)

    return pl.pallas_call(
        body,
        out_shape=jax.ShapeDtypeStruct((m, d), jnp.float32),
        in_specs=[
            pl.BlockSpec(memory_space=pltpu.VMEM),
            pl.BlockSpec(memory_space=pltpu.VMEM),
        ],
        out_specs=pl.BlockSpec(memory_space=pltpu.VMEM),
        scratch_shapes=[
            pltpu.VMEM((J - 1, NSUB, sa, d), jnp.float32),
            pltpu.VMEM((J - 1, NSUB, sa, d), jnp.float32),
            pltpu.VMEM((P - 1, NSUB, sb, d), jnp.float32),
            pltpu.VMEM((P - 1, NSUB, sb, d), jnp.float32),
            pltpu.SemaphoreType.DMA((NSUB,)),
            pltpu.SemaphoreType.DMA((NSUB,)),
            pltpu.SemaphoreType.DMA((NSUB,)),
            pltpu.SemaphoreType.DMA((NSUB,)),
            pltpu.SemaphoreType.DMA((2 * (J - 1), NSUB)),
            pltpu.SemaphoreType.DMA((2 * (J - 1), NSUB)),
            pltpu.SemaphoreType.DMA((2 * (P - 1), NSUB)),
            pltpu.SemaphoreType.DMA((2 * (P - 1), NSUB)),
        ],
        compiler_params=pltpu.CompilerParams(collective_id=0),
    )(dy, W)
